# baseline (speedup 1.0000x reference)
import functools

import jax
import jax.numpy as jnp
import numpy as np
from jax.experimental import pallas as pl
from jax.experimental.pallas import tpu as pltpu

N_BINS = 15
BLOCK_ROWS = 512

_BOUNDARIES = np.linspace(0.0, 1.0, N_BINS + 1)


def _ece_body(logits_ref, labels_ref, ece_ref, cnt_ref, sconf_ref, sacc_ref,
              *, n_total):
    i = pl.program_id(0)

    @pl.when(i == 0)
    def _init():
        cnt_ref[...] = jnp.zeros_like(cnt_ref)
        sconf_ref[...] = jnp.zeros_like(sconf_ref)
        sacc_ref[...] = jnp.zeros_like(sacc_ref)

    x = logits_ref[...]
    m = jnp.max(x, axis=1, keepdims=True)
    s = jnp.sum(jnp.exp(x - m), axis=1)
    conf = 1.0 / s
    pred = jnp.argmax(x, axis=1).astype(jnp.int32)
    acc = (pred == labels_ref[...]).astype(jnp.float32)

    k = jax.lax.broadcasted_iota(jnp.int32, (1, N_BINS), 1).astype(jnp.float32)
    lo = k / N_BINS
    hi = (k + 1.0) / N_BINS
    c2 = conf[:, None]
    mask = ((c2 > lo) & (c2 <= hi)).astype(jnp.float32)
    cnt_ref[...] += jnp.sum(mask, axis=0)
    sconf_ref[...] += jnp.sum(c2 * mask, axis=0)
    sacc_ref[...] += jnp.sum(acc[:, None] * mask, axis=0)

    @pl.when(i == pl.num_programs(0) - 1)
    def _finish():
        cnt = cnt_ref[...]
        safe = jnp.maximum(cnt, 1.0)
        gap = jnp.abs(sconf_ref[...] / safe - sacc_ref[...] / safe) * (cnt / n_total)
        gap = jnp.where(cnt > 0, gap, 0.0)
        ece_ref[...] = jnp.sum(gap, keepdims=True)


def kernel(logits, labels):
    n, c = logits.shape
    labels = labels.astype(jnp.int32)
    grid = (n // BLOCK_ROWS,)
    return pl.pallas_call(
        functools.partial(_ece_body, n_total=float(n)),
        grid=grid,
        in_specs=[
            pl.BlockSpec((BLOCK_ROWS, c), lambda i: (i, 0)),
            pl.BlockSpec((BLOCK_ROWS,), lambda i: (i,)),
        ],
        out_specs=pl.BlockSpec((1,), lambda i: (0,)),
        out_shape=jax.ShapeDtypeStruct((1,), jnp.float32),
        scratch_shapes=[
            pltpu.VMEM((N_BINS,), jnp.float32),
            pltpu.VMEM((N_BINS,), jnp.float32),
            pltpu.VMEM((N_BINS,), jnp.float32),
        ],
    )(logits, labels)


# speedup vs baseline: 1.1309x; 1.1309x over previous
import functools

import jax
import jax.numpy as jnp
import numpy as np
from jax.experimental import pallas as pl
from jax.experimental.pallas import tpu as pltpu

N_BINS = 15
BLOCK_ROWS = 512

_BOUNDARIES = np.linspace(0.0, 1.0, N_BINS + 1)


def _ece_body(logits_ref, labels_ref, ece_ref, cnt_ref, sconf_ref, sacc_ref,
              *, n_total):
    i = pl.program_id(0)

    @pl.when(i == 0)
    def _init():
        cnt_ref[...] = jnp.zeros_like(cnt_ref)
        sconf_ref[...] = jnp.zeros_like(sconf_ref)
        sacc_ref[...] = jnp.zeros_like(sacc_ref)

    x = logits_ref[...]
    m = jnp.max(x, axis=1, keepdims=True)
    s = jnp.sum(jnp.exp(x - m), axis=1)
    conf = 1.0 / s
    col = jax.lax.broadcasted_iota(jnp.int32, x.shape, 1)
    hit = (x == m) & (col == labels_ref[...][:, None])
    acc = jnp.max(hit.astype(jnp.float32), axis=1)

    k = jax.lax.broadcasted_iota(jnp.int32, (1, N_BINS), 1).astype(jnp.float32)
    lo = k / N_BINS
    hi = (k + 1.0) / N_BINS
    c2 = conf[:, None]
    mask = ((c2 > lo) & (c2 <= hi)).astype(jnp.float32)
    cnt_ref[...] += jnp.sum(mask, axis=0)
    sconf_ref[...] += jnp.sum(c2 * mask, axis=0)
    sacc_ref[...] += jnp.sum(acc[:, None] * mask, axis=0)

    @pl.when(i == pl.num_programs(0) - 1)
    def _finish():
        cnt = cnt_ref[...]
        safe = jnp.maximum(cnt, 1.0)
        gap = jnp.abs(sconf_ref[...] / safe - sacc_ref[...] / safe) * (cnt / n_total)
        gap = jnp.where(cnt > 0, gap, 0.0)
        ece_ref[...] = jnp.sum(gap, keepdims=True)


def kernel(logits, labels):
    n, c = logits.shape
    labels = labels.astype(jnp.int32)
    grid = (n // BLOCK_ROWS,)
    return pl.pallas_call(
        functools.partial(_ece_body, n_total=float(n)),
        grid=grid,
        in_specs=[
            pl.BlockSpec((BLOCK_ROWS, c), lambda i: (i, 0)),
            pl.BlockSpec((BLOCK_ROWS,), lambda i: (i,)),
        ],
        out_specs=pl.BlockSpec((1,), lambda i: (0,)),
        out_shape=jax.ShapeDtypeStruct((1,), jnp.float32),
        scratch_shapes=[
            pltpu.VMEM((N_BINS,), jnp.float32),
            pltpu.VMEM((N_BINS,), jnp.float32),
            pltpu.VMEM((N_BINS,), jnp.float32),
        ],
    )(logits, labels)


# speedup vs baseline: 1.2334x; 1.0907x over previous
import functools

import jax
import jax.numpy as jnp
import numpy as np
from jax.experimental import pallas as pl
from jax.experimental.pallas import tpu as pltpu

N_BINS = 15
BLOCK_ROWS = 1024

_BOUNDARIES = np.linspace(0.0, 1.0, N_BINS + 1)


def _ece_body(logits_ref, labels_ref, ece_ref, cnt_ref, sconf_ref, sacc_ref,
              *, n_total):
    i = pl.program_id(0)

    @pl.when(i == 0)
    def _init():
        cnt_ref[...] = jnp.zeros_like(cnt_ref)
        sconf_ref[...] = jnp.zeros_like(sconf_ref)
        sacc_ref[...] = jnp.zeros_like(sacc_ref)

    x = logits_ref[...]
    m = jnp.max(x, axis=1, keepdims=True)
    s = jnp.sum(jnp.exp(x - m), axis=1)
    conf = 1.0 / s
    col = jax.lax.broadcasted_iota(jnp.int32, x.shape, 1)
    hit = (x == m) & (col == labels_ref[...][:, None])
    acc = jnp.max(hit.astype(jnp.float32), axis=1)

    k = jax.lax.broadcasted_iota(jnp.int32, (1, N_BINS), 1).astype(jnp.float32)
    lo = k / N_BINS
    hi = (k + 1.0) / N_BINS
    c2 = conf[:, None]
    mask = ((c2 > lo) & (c2 <= hi)).astype(jnp.float32)
    cnt_ref[...] += jnp.sum(mask, axis=0)
    sconf_ref[...] += jnp.sum(c2 * mask, axis=0)
    sacc_ref[...] += jnp.sum(acc[:, None] * mask, axis=0)

    @pl.when(i == pl.num_programs(0) - 1)
    def _finish():
        cnt = cnt_ref[...]
        safe = jnp.maximum(cnt, 1.0)
        gap = jnp.abs(sconf_ref[...] / safe - sacc_ref[...] / safe) * (cnt / n_total)
        gap = jnp.where(cnt > 0, gap, 0.0)
        ece_ref[...] = jnp.sum(gap, keepdims=True)


def kernel(logits, labels):
    n, c = logits.shape
    labels = labels.astype(jnp.int32)
    grid = (n // BLOCK_ROWS,)
    return pl.pallas_call(
        functools.partial(_ece_body, n_total=float(n)),
        grid=grid,
        in_specs=[
            pl.BlockSpec((BLOCK_ROWS, c), lambda i: (i, 0)),
            pl.BlockSpec((BLOCK_ROWS,), lambda i: (i,)),
        ],
        out_specs=pl.BlockSpec((1,), lambda i: (0,)),
        out_shape=jax.ShapeDtypeStruct((1,), jnp.float32),
        scratch_shapes=[
            pltpu.VMEM((N_BINS,), jnp.float32),
            pltpu.VMEM((N_BINS,), jnp.float32),
            pltpu.VMEM((N_BINS,), jnp.float32),
        ],
    )(logits, labels)


# speedup vs baseline: 1.2994x; 1.0535x over previous
import functools

import jax
import jax.numpy as jnp
import numpy as np
from jax.experimental import pallas as pl
from jax.experimental.pallas import tpu as pltpu

N_BINS = 15
BLOCK_ROWS = 2048

_BOUNDARIES = np.linspace(0.0, 1.0, N_BINS + 1)


def _ece_body(logits_ref, labels_ref, ece_ref, cnt_ref, sconf_ref, sacc_ref,
              *, n_total):
    i = pl.program_id(0)

    @pl.when(i == 0)
    def _init():
        cnt_ref[...] = jnp.zeros_like(cnt_ref)
        sconf_ref[...] = jnp.zeros_like(sconf_ref)
        sacc_ref[...] = jnp.zeros_like(sacc_ref)

    x = logits_ref[...]
    m = jnp.max(x, axis=1, keepdims=True)
    s = jnp.sum(jnp.exp(x - m), axis=1)
    conf = 1.0 / s
    col = jax.lax.broadcasted_iota(jnp.int32, x.shape, 1)
    hit = (x == m) & (col == labels_ref[...][:, None])
    acc = jnp.max(hit.astype(jnp.float32), axis=1)

    k = jax.lax.broadcasted_iota(jnp.int32, (1, N_BINS), 1).astype(jnp.float32)
    lo = k / N_BINS
    hi = (k + 1.0) / N_BINS
    c2 = conf[:, None]
    mask = ((c2 > lo) & (c2 <= hi)).astype(jnp.float32)
    cnt_ref[...] += jnp.sum(mask, axis=0)
    sconf_ref[...] += jnp.sum(c2 * mask, axis=0)
    sacc_ref[...] += jnp.sum(acc[:, None] * mask, axis=0)

    @pl.when(i == pl.num_programs(0) - 1)
    def _finish():
        cnt = cnt_ref[...]
        safe = jnp.maximum(cnt, 1.0)
        gap = jnp.abs(sconf_ref[...] / safe - sacc_ref[...] / safe) * (cnt / n_total)
        gap = jnp.where(cnt > 0, gap, 0.0)
        ece_ref[...] = jnp.sum(gap, keepdims=True)


def kernel(logits, labels):
    n, c = logits.shape
    labels = labels.astype(jnp.int32)
    grid = (n // BLOCK_ROWS,)
    return pl.pallas_call(
        functools.partial(_ece_body, n_total=float(n)),
        grid=grid,
        in_specs=[
            pl.BlockSpec((BLOCK_ROWS, c), lambda i: (i, 0)),
            pl.BlockSpec((BLOCK_ROWS,), lambda i: (i,)),
        ],
        out_specs=pl.BlockSpec((1,), lambda i: (0,)),
        out_shape=jax.ShapeDtypeStruct((1,), jnp.float32),
        scratch_shapes=[
            pltpu.VMEM((N_BINS,), jnp.float32),
            pltpu.VMEM((N_BINS,), jnp.float32),
            pltpu.VMEM((N_BINS,), jnp.float32),
        ],
    )(logits, labels)


# speedup vs baseline: 1.3020x; 1.0020x over previous
import functools

import jax
import jax.numpy as jnp
import numpy as np
from jax.experimental import pallas as pl
from jax.experimental.pallas import tpu as pltpu

N_BINS = 15
BLOCK_ROWS = 4096

_BOUNDARIES = np.linspace(0.0, 1.0, N_BINS + 1)


def _ece_body(logits_ref, labels_ref, ece_ref, cnt_ref, sconf_ref, sacc_ref,
              *, n_total):
    i = pl.program_id(0)

    @pl.when(i == 0)
    def _init():
        cnt_ref[...] = jnp.zeros_like(cnt_ref)
        sconf_ref[...] = jnp.zeros_like(sconf_ref)
        sacc_ref[...] = jnp.zeros_like(sacc_ref)

    x = logits_ref[...]
    m = jnp.max(x, axis=1, keepdims=True)
    s = jnp.sum(jnp.exp(x - m), axis=1)
    conf = 1.0 / s
    col = jax.lax.broadcasted_iota(jnp.int32, x.shape, 1)
    hit = (x == m) & (col == labels_ref[...][:, None])
    acc = jnp.max(hit.astype(jnp.float32), axis=1)

    k = jax.lax.broadcasted_iota(jnp.int32, (1, N_BINS), 1).astype(jnp.float32)
    lo = k / N_BINS
    hi = (k + 1.0) / N_BINS
    c2 = conf[:, None]
    mask = ((c2 > lo) & (c2 <= hi)).astype(jnp.float32)
    cnt_ref[...] += jnp.sum(mask, axis=0)
    sconf_ref[...] += jnp.sum(c2 * mask, axis=0)
    sacc_ref[...] += jnp.sum(acc[:, None] * mask, axis=0)

    @pl.when(i == pl.num_programs(0) - 1)
    def _finish():
        cnt = cnt_ref[...]
        safe = jnp.maximum(cnt, 1.0)
        gap = jnp.abs(sconf_ref[...] / safe - sacc_ref[...] / safe) * (cnt / n_total)
        gap = jnp.where(cnt > 0, gap, 0.0)
        ece_ref[...] = jnp.sum(gap, keepdims=True)


def kernel(logits, labels):
    n, c = logits.shape
    labels = labels.astype(jnp.int32)
    grid = (n // BLOCK_ROWS,)
    return pl.pallas_call(
        functools.partial(_ece_body, n_total=float(n)),
        grid=grid,
        in_specs=[
            pl.BlockSpec((BLOCK_ROWS, c), lambda i: (i, 0)),
            pl.BlockSpec((BLOCK_ROWS,), lambda i: (i,)),
        ],
        out_specs=pl.BlockSpec((1,), lambda i: (0,)),
        out_shape=jax.ShapeDtypeStruct((1,), jnp.float32),
        scratch_shapes=[
            pltpu.VMEM((N_BINS,), jnp.float32),
            pltpu.VMEM((N_BINS,), jnp.float32),
            pltpu.VMEM((N_BINS,), jnp.float32),
        ],
    )(logits, labels)
